# Initial kernel scaffold; baseline (speedup 1.0000x reference)
import jax
import jax.numpy as jnp
from jax.experimental import pallas as pl


def kernel(mem, emb_table, gamma, beta):
    raise NotImplementedError("write your pallas kernel here")


# speedup vs baseline: 1.9336x; 1.9336x over previous
import functools

import jax
import jax.numpy as jnp
from jax.experimental import pallas as pl

MEM_LENGTH = 8192
HIDDEN = 768
BATCH = 4

_BL = 512


def _ln_body(mem_ref, emb_ref, gamma_ref, beta_ref, out_ref):
    x = mem_ref[0] + emb_ref[...]
    mean = jnp.mean(x, axis=-1, keepdims=True)
    xc = x - mean
    var = jnp.mean(xc * xc, axis=-1, keepdims=True)
    inv = jax.lax.rsqrt(var + 1e-5)
    out_ref[0] = xc * inv * gamma_ref[...] + beta_ref[...]


@functools.partial(jax.jit)
def kernel(mem, emb_table, gamma, beta):
    grid = (BATCH, MEM_LENGTH // _BL)
    return pl.pallas_call(
        _ln_body,
        grid=grid,
        in_specs=[
            pl.BlockSpec((1, _BL, HIDDEN), lambda b, i: (b, i, 0)),
            pl.BlockSpec((_BL, HIDDEN), lambda b, i: (i, 0)),
            pl.BlockSpec((HIDDEN,), lambda b, i: (0,)),
            pl.BlockSpec((HIDDEN,), lambda b, i: (0,)),
        ],
        out_specs=pl.BlockSpec((1, _BL, HIDDEN), lambda b, i: (b, i, 0)),
        out_shape=jax.ShapeDtypeStruct((BATCH, MEM_LENGTH, HIDDEN), jnp.float32),
    )(mem, emb_table, gamma, beta)
